# baseline (speedup 1.0000x reference)
import functools

import jax
import jax.numpy as jnp
from jax import lax
from jax.experimental import pallas as pl
from jax.experimental.pallas import tpu as pltpu
from jax.experimental.pallas import tpu_sc as plsc


def _feats_body(f_in, f_out):
    f_out[...] = f_in[...]


def _copy_feats(feats):
    n, d = feats.shape
    grid = 5
    return pl.pallas_call(
        _feats_body,
        grid=(grid,),
        in_specs=[pl.BlockSpec((n // grid, d), lambda i: (i, 0))],
        out_specs=pl.BlockSpec((n // grid, d), lambda i: (i, 0)),
        out_shape=jax.ShapeDtypeStruct(feats.shape, feats.dtype),
        compiler_params=pltpu.CompilerParams(
            dimension_semantics=("arbitrary",),
        ),
    )(feats)


def _make_sc_copy(ei_shape, ea_shape, ei_dtype, ea_dtype, nc, ns):
    nw = nc * ns
    ei_rows_w = ei_shape[0] // nw
    ea_rows_w = ea_shape[0] // nw
    mesh = plsc.VectorSubcoreMesh(core_axis_name="c", subcore_axis_name="s")

    @functools.partial(
        pl.kernel,
        mesh=mesh,
        out_type=[
            jax.ShapeDtypeStruct(ei_shape, ei_dtype),
            jax.ShapeDtypeStruct(ea_shape, ea_dtype),
        ],
        scratch_types=[
            pltpu.VMEM((1, ei_shape[1]), ei_dtype),
            pltpu.VMEM((1, ei_shape[1]), ei_dtype),
            pltpu.VMEM((1, ea_shape[1]), ea_dtype),
            pltpu.VMEM((1, ea_shape[1]), ea_dtype),
            pltpu.SemaphoreType.DMA((2, 2)),
            pltpu.SemaphoreType.DMA((2, 2)),
        ],
    )
    def sc_copy(ei_hbm, ea_hbm, ei_out, ea_out, ei_v0, ei_v1, ea_v0, ea_v1, in_sem, out_sem):
        wid = lax.axis_index("s") * nc + lax.axis_index("c")

        def copy_array(src, dst, bufs, rows_w, arr):
            base = wid * rows_w

            def start_in(j, b):
                pltpu.async_copy(src.at[pl.ds(base + j, 1)], bufs[b], in_sem.at[b, arr])

            def wait_in(b):
                pltpu.make_async_copy(src.at[pl.ds(base, 1)], bufs[b], in_sem.at[b, arr]).wait()

            def start_out(j, b):
                pltpu.async_copy(bufs[b], dst.at[pl.ds(base + j, 1)], out_sem.at[b, arr])

            def wait_out(b):
                pltpu.make_async_copy(bufs[b], dst.at[pl.ds(base, 1)], out_sem.at[b, arr]).wait()

            start_in(0, 0)
            if rows_w > 1:
                start_in(1, 1)
            for j in range(rows_w):
                b = j % 2
                wait_in(b)
                start_out(j, b)
                if j + 2 < rows_w:
                    wait_out(b)
                    start_in(j + 2, b)
            wait_out((rows_w - 1) % 2)
            if rows_w > 1:
                wait_out(rows_w % 2)

        copy_array(ei_hbm, ei_out, (ei_v0, ei_v1), ei_rows_w, 0)
        copy_array(ea_hbm, ea_out, (ea_v0, ea_v1), ea_rows_w, 1)

    return sc_copy


def kernel(feats, edge_index, edge_attr):
    e, ik = edge_index.shape
    _, ak = edge_attr.shape

    zi = lax.optimization_barrier(jnp.zeros((), edge_index.dtype))
    of = lax.optimization_barrier(jnp.ones((), edge_attr.dtype))

    ei2 = jnp.bitwise_xor(edge_index.reshape(64, (e * ik) // 64), zi)
    ea2 = edge_attr.reshape(320, (e * ak) // 320) * of

    info = plsc.get_sparse_core_info()
    sc_copy = _make_sc_copy(ei2.shape, ea2.shape, ei2.dtype, ea2.dtype,
                            info.num_cores, info.num_subcores)
    ei_o, ea_o = sc_copy(ei2, ea2)
    f_o = _copy_feats(feats)

    ei_final = jnp.bitwise_xor(ei_o.reshape(e, ik), zi)
    ea_final = ea_o.reshape(e, ak) * of
    return (f_o, ei_final, ea_final)
